# baseline (speedup 1.0000x reference)
import functools

import jax
import jax.numpy as jnp
from jax import lax
from jax.experimental import pallas as pl
from jax.experimental.pallas import tpu as pltpu
from jax.experimental.pallas import tpu_sc as plsc

N_TOK = 65536
K = 32
KP = 128
V = 8192
M = 256
NB = N_TOK // M


def _tc_body(z_ref, wt_ref, idx_ref):
    wt = wt_ref[...]
    w2 = jnp.sum(wt * wt, axis=0, keepdims=True)
    iota = lax.broadcasted_iota(jnp.int32, (V, 1), 0).astype(jnp.float32)
    s = jnp.dot(z_ref[...], wt, preferred_element_type=jnp.float32,
                precision=lax.Precision.HIGHEST)
    d = w2 - (s + s)
    m = jnp.min(d, axis=1, keepdims=True)
    onehot = jnp.where(d == m, 1.0, 0.0)
    idxf = jnp.dot(onehot, iota, preferred_element_type=jnp.float32,
                   precision=lax.Precision.HIGHEST)
    idx_ref[...] = idxf.astype(jnp.int32)


def _tc_argmin(z_flat, wt):
    return pl.pallas_call(
        _tc_body,
        grid=(NB,),
        in_specs=[
            pl.BlockSpec((M, K), lambda i: (i, 0)),
            pl.BlockSpec((K, V), lambda i: (0, 0)),
        ],
        out_specs=pl.BlockSpec((M, 1), lambda i: (i, 0)),
        out_shape=jax.ShapeDtypeStruct((N_TOK, 1), jnp.int32),
    )(z_flat, wt)


@functools.cache
def _make_sc_gather():
    info = plsc.get_sparse_core_info()
    nc, ns = info.num_cores, info.num_subcores
    nw = nc * ns
    b_per_w = N_TOK // nw
    chunk = 128
    n_chunks = b_per_w // chunk
    mesh = plsc.VectorSubcoreMesh(core_axis_name="c", subcore_axis_name="s")

    @functools.partial(
        pl.kernel,
        mesh=mesh,
        out_type=jax.ShapeDtypeStruct((N_TOK, KP), jnp.float32),
        scratch_types=[
            pltpu.VMEM((n_chunks, chunk), jnp.int32),
            pltpu.VMEM((chunk, KP), jnp.float32),
            pltpu.SemaphoreType.DMA,
        ],
    )
    def gather(table_hbm, idx_hbm, out_hbm, idx_v, rows_v, sem):
        wid = lax.axis_index("s") * nc + lax.axis_index("c")
        base = wid * b_per_w
        pltpu.sync_copy(idx_hbm.at[wid], idx_v)
        for j in range(n_chunks):
            pltpu.async_copy(table_hbm.at[idx_v.at[j]], rows_v, sem).wait()
            pltpu.sync_copy(rows_v, out_hbm.at[pl.ds(base + j * chunk, chunk)])

    return gather, nw, b_per_w, chunk


def kernel(z, weight):
    z_flat = z.reshape(-1, K)
    wt = weight.T
    idx2 = _tc_argmin(z_flat, wt)
    indices = idx2.reshape(N_TOK)
    sc_gather, nw, b_per_w, chunk = _make_sc_gather()
    table = jnp.pad(weight, ((0, 0), (0, KP - K)))
    idx3 = indices.reshape(nw, b_per_w // chunk, chunk)
    quantized = sc_gather(table, idx3)[:, :K]
    return (quantized, indices)


# speedup vs baseline: 8.4907x; 8.4907x over previous
import functools

import jax
import jax.numpy as jnp
from jax import lax
from jax.experimental import pallas as pl
from jax.experimental.pallas import tpu as pltpu
from jax.experimental.pallas import tpu_sc as plsc

N_TOK = 65536
K = 32
KP = 128
V = 8192
M = 256
NB = N_TOK // M


def _tc_body(z_ref, wt_ref, idx_ref):
    wt = wt_ref[...]
    w2 = jnp.sum(wt * wt, axis=0, keepdims=True)
    s = jnp.dot(z_ref[...], wt, preferred_element_type=jnp.float32)
    d = w2 - (s + s)
    m = jnp.min(d, axis=1, keepdims=True)
    iota_row = lax.broadcasted_iota(jnp.int32, (M, V), 1)
    cand = jnp.where(d == m, iota_row, jnp.int32(2 ** 30))
    idx_ref[...] = jnp.min(cand, axis=1).reshape(M, 1)


def _tc_argmin(z_flat, wt):
    return pl.pallas_call(
        _tc_body,
        grid=(NB,),
        in_specs=[
            pl.BlockSpec((M, K), lambda i: (i, 0)),
            pl.BlockSpec((K, V), lambda i: (0, 0)),
        ],
        out_specs=pl.BlockSpec((M, 1), lambda i: (i, 0)),
        out_shape=jax.ShapeDtypeStruct((N_TOK, 1), jnp.int32),
    )(z_flat, wt)


@functools.cache
def _make_sc_gather():
    info = plsc.get_sparse_core_info()
    nc, ns = info.num_cores, info.num_subcores
    nw = nc * ns
    b_per_w = N_TOK // nw
    chunk = 128
    n_chunks = b_per_w // chunk
    mesh = plsc.VectorSubcoreMesh(core_axis_name="c", subcore_axis_name="s")

    @functools.partial(
        pl.kernel,
        mesh=mesh,
        out_type=jax.ShapeDtypeStruct((N_TOK, KP), jnp.float32),
        scratch_types=[
            pltpu.VMEM((n_chunks, chunk), jnp.int32),
            pltpu.VMEM((chunk, KP), jnp.float32),
            pltpu.SemaphoreType.DMA,
        ],
    )
    def gather(table_hbm, idx_hbm, out_hbm, idx_v, rows_v, sem):
        wid = lax.axis_index("s") * nc + lax.axis_index("c")
        base = wid * b_per_w
        pltpu.sync_copy(idx_hbm.at[wid], idx_v)
        for j in range(n_chunks):
            pltpu.async_copy(table_hbm.at[idx_v.at[j]], rows_v, sem).wait()
            pltpu.sync_copy(rows_v, out_hbm.at[pl.ds(base + j * chunk, chunk)])

    return gather, nw, b_per_w, chunk


def kernel(z, weight):
    z_flat = z.reshape(-1, K)
    wt = weight.T
    idx2 = _tc_argmin(z_flat, wt)
    indices = idx2.reshape(N_TOK)
    sc_gather, nw, b_per_w, chunk = _make_sc_gather()
    table = jnp.pad(weight, ((0, 0), (0, KP - K)))
    idx3 = indices.reshape(nw, b_per_w // chunk, chunk)
    quantized = sc_gather(table, idx3)[:, :K]
    return (quantized, indices)


# speedup vs baseline: 8.8964x; 1.0478x over previous
import functools

import jax
import jax.numpy as jnp
from jax import lax
from jax.experimental import pallas as pl
from jax.experimental.pallas import tpu as pltpu
from jax.experimental.pallas import tpu_sc as plsc

N_TOK = 65536
K = 32
KP = 128
V = 8192
M = 512
NB = N_TOK // M


def _tc_body(z_ref, wt_ref, idx_ref):
    wt = wt_ref[...]
    w2 = jnp.sum(wt * wt, axis=0, keepdims=True)
    s = jnp.dot(z_ref[...], wt, preferred_element_type=jnp.float32)
    d = w2 - (s + s)
    m = jnp.min(d, axis=1, keepdims=True)
    iota_row = lax.broadcasted_iota(jnp.int32, (M, V), 1)
    cand = jnp.where(d == m, iota_row, jnp.int32(2 ** 30))
    idx_ref[...] = jnp.min(cand, axis=1).reshape(M, 1)


def _tc_argmin(z_flat, wt):
    return pl.pallas_call(
        _tc_body,
        grid=(NB,),
        in_specs=[
            pl.BlockSpec((M, K), lambda i: (i, 0)),
            pl.BlockSpec((K, V), lambda i: (0, 0)),
        ],
        out_specs=pl.BlockSpec((M, 1), lambda i: (i, 0)),
        out_shape=jax.ShapeDtypeStruct((N_TOK, 1), jnp.int32),
    )(z_flat, wt)


@functools.cache
def _make_sc_gather():
    info = plsc.get_sparse_core_info()
    nc, ns = info.num_cores, info.num_subcores
    nw = nc * ns
    b_per_w = N_TOK // nw
    chunk = 128
    n_chunks = b_per_w // chunk
    mesh = plsc.VectorSubcoreMesh(core_axis_name="c", subcore_axis_name="s")

    @functools.partial(
        pl.kernel,
        mesh=mesh,
        out_type=jax.ShapeDtypeStruct((N_TOK, KP), jnp.float32),
        scratch_types=[
            pltpu.VMEM((n_chunks, chunk), jnp.int32),
            pltpu.VMEM((chunk, KP), jnp.float32),
            pltpu.SemaphoreType.DMA,
        ],
    )
    def gather(table_hbm, idx_hbm, out_hbm, idx_v, rows_v, sem):
        wid = lax.axis_index("s") * nc + lax.axis_index("c")
        base = wid * b_per_w
        pltpu.sync_copy(idx_hbm.at[wid], idx_v)
        for j in range(n_chunks):
            pltpu.async_copy(table_hbm.at[idx_v.at[j]], rows_v, sem).wait()
            pltpu.sync_copy(rows_v, out_hbm.at[pl.ds(base + j * chunk, chunk)])

    return gather, nw, b_per_w, chunk


def kernel(z, weight):
    z_flat = z.reshape(-1, K)
    wt = weight.T
    idx2 = _tc_argmin(z_flat, wt)
    indices = idx2.reshape(N_TOK)
    sc_gather, nw, b_per_w, chunk = _make_sc_gather()
    table = jnp.pad(weight, ((0, 0), (0, KP - K)))
    idx3 = indices.reshape(nw, b_per_w // chunk, chunk)
    quantized = sc_gather(table, idx3)[:, :K]
    return (quantized, indices)
